# baseline (speedup 1.0000x reference)
import functools

import jax
import jax.numpy as jnp
from jax import lax
from jax.experimental import pallas as pl
from jax.experimental.pallas import tpu as pltpu
from jax.experimental.pallas import tpu_sc as plsc

N = 50000
E = 800000
G = 64
NP = 50176
EP = 802816
NB = NP // 512
EB = EP // 512
NPT = NP // 16

C1 = 0.9864523801332455
C2 = -0.40368039920027776
C3 = 0.1105132845506506


def _log1p16(t):
    return t * (C1 + t * (C2 + t * C3))


def _gate16(a, b):
    ta = jnp.exp(-jnp.abs(a))
    tb = jnp.exp(-jnp.abs(b))
    sp_b = jnp.maximum(b, 0.0) + _log1p16(tb)
    sig_a = jnp.exp(jnp.minimum(a, 0.0) - _log1p16(ta))
    return sig_a * sp_b


def _sc_edge_layer(tw, cpt, ebase_c, idxoff_c, eoff_c, d_tbl, s_tbl, e_arr,
                   dg, sg, ds, zz):
    mw = tw // 2
    ck = 64
    mesh = plsc.VectorSubcoreMesh(core_axis_name="c", subcore_axis_name="s")

    @functools.partial(
        pl.kernel,
        mesh=mesh,
        compiler_params=pltpu.CompilerParams(use_tc_tiling_on_sc=False),
        out_type=jax.ShapeDtypeStruct((2 * NP, mw), jnp.float32),
        scratch_types=[
            pltpu.VMEM((3, ck), jnp.int32),
            pltpu.VMEM((3, ck), jnp.int32),
            pltpu.VMEM((3, ck), jnp.int32),
            pltpu.VMEM((2, ck, tw), jnp.float32),
            pltpu.VMEM((2, ck, tw), jnp.float32),
            pltpu.VMEM((2, ck, tw // 2), jnp.int32),
            pltpu.VMEM((2, ck, mw), jnp.float32),
            pltpu.VMEM_SHARED((NP, mw), jnp.float32),
            pltpu.SemaphoreType.DMA,
            pltpu.SemaphoreType.DMA,
            pltpu.SemaphoreType.DMA,
            pltpu.SemaphoreType.DMA,
            pltpu.SemaphoreType.DMA,
        ],
    )
    def k(d_hbm, s_hbm, e_hbm, dg_hbm, sg_hbm, ds_hbm, zz_hbm, out_hbm,
          dgi_v, sgi_v, dsi_v, rows_d, rows_s, rows_e, m_v, acc,
          sem_d, sem_s, sem_e, sem_i, sem_m):
        c = lax.axis_index("c")
        s = lax.axis_index("s")
        pltpu.sync_copy(zz_hbm, acc.at[pl.ds(s * NPT, NPT)])
        plsc.subcore_barrier()

        base = ebase_c * c + s * (cpt * ck)

        def idx_issue(i):
            e0 = base + i * ck
            eg = idxoff_c * c + e0
            p = lax.rem(i, 3)
            pltpu.async_copy(dg_hbm.at[pl.ds(eg, ck)], dgi_v.at[p], sem_i)
            pltpu.async_copy(sg_hbm.at[pl.ds(eg, ck)], sgi_v.at[p], sem_i)
            pltpu.async_copy(ds_hbm.at[pl.ds(e0, ck)], dsi_v.at[p], sem_i)

        def idx_wait(i):
            p = lax.rem(i, 3)
            pltpu.make_async_copy(dg_hbm.at[pl.ds(0, ck)], dgi_v.at[p], sem_i).wait()
            pltpu.make_async_copy(sg_hbm.at[pl.ds(0, ck)], sgi_v.at[p], sem_i).wait()
            pltpu.make_async_copy(ds_hbm.at[pl.ds(0, ck)], dsi_v.at[p], sem_i).wait()

        def rows_issue(i):
            e0 = base + i * ck
            p = lax.rem(i, 2)
            pi = lax.rem(i, 3)
            pltpu.async_copy(d_hbm.at[dgi_v.at[pi]], rows_d.at[p], sem_d)
            pltpu.async_copy(s_hbm.at[sgi_v.at[pi]], rows_s.at[p], sem_s)
            pltpu.async_copy(e_hbm.at[pl.ds(eoff_c * c + e0, ck)],
                             rows_e.at[p], sem_e)

        def rows_wait(i):
            p = lax.rem(i, 2)
            pltpu.make_async_copy(e_hbm.at[pl.ds(0, ck)], rows_d.at[p], sem_d).wait()
            pltpu.make_async_copy(e_hbm.at[pl.ds(0, ck)], rows_s.at[p], sem_s).wait()
            pltpu.make_async_copy(e_hbm.at[pl.ds(0, ck)], rows_e.at[p], sem_e).wait()

        idx_issue(0)
        idx_wait(0)
        rows_issue(0)
        idx_issue(1)

        def chunk(i, carry):
            rows_wait(i)

            @pl.when(i > 0)
            def _w_sc():
                pltpu.make_async_copy(
                    zz_hbm.at[pl.ds(0, ck)], m_v.at[0], sem_m).wait()

            @pl.when(i + 1 < cpt)
            def _pf_rows():
                idx_wait(i + 1)
                rows_issue(i + 1)

            @pl.when(i + 2 < cpt)
            def _pf_idx():
                idx_issue(i + 2)

            p = lax.rem(i, 2)
            pi = lax.rem(i, 3)

            @plsc.parallel_loop(0, ck, unroll=8)
            def row(r):
                def unpk(w):
                    lo = lax.bitcast_convert_type(
                        lax.shift_left(w, 16), jnp.float32)
                    hi = lax.bitcast_convert_type(
                        lax.bitwise_and(w, jnp.int32(-65536)), jnp.float32)
                    return lo, hi

                if tw == 64:
                    ea = unpk(rows_e[p, r, pl.ds(0, 16)])
                    eb = unpk(rows_e[p, r, pl.ds(16, 16)])
                else:
                    lo, hi = unpk(rows_e[p, r, :])
                    ea, eb = (lo,), (hi,)
                for j in range(tw // 32):
                    oa = 16 * j
                    ob = mw + 16 * j
                    a = (rows_d[p, r, pl.ds(oa, 16)] + rows_s[p, r, pl.ds(oa, 16)]
                         + ea[j])
                    b = (rows_d[p, r, pl.ds(ob, 16)] + rows_s[p, r, pl.ds(ob, 16)]
                         + eb[j])
                    m_v[p, r, pl.ds(oa, 16)] = _gate16(a, b)
            pltpu.async_copy(m_v.at[p], acc.at[dsi_v.at[pi]], sem_m, add=True)
            return carry

        lax.fori_loop(0, cpt, chunk, 0)
        pltpu.make_async_copy(zz_hbm.at[pl.ds(0, ck)], m_v.at[0], sem_m).wait()
        plsc.subcore_barrier()
        pltpu.sync_copy(acc.at[pl.ds(s * NPT, NPT)],
                        out_hbm.at[pl.ds(c * NP + s * NPT, NPT)])

    return k(d_tbl, s_tbl, e_arr, dg, sg, ds, zz)


def _dot(a, b):
    return jax.lax.dot_general(a, b, (((1,), (0,)), ((), ())),
                               preferred_element_type=jnp.float32)


def _node_prep1(x_pad, wd1t, ws1t):
    def body(x_ref, wd_ref, ws_ref, d_ref, s_ref):
        xb = x_ref[...]
        d_ref[...] = _dot(xb, wd_ref[...])
        s_ref[...] = _dot(xb, ws_ref[...])

    return pl.pallas_call(
        body,
        grid=(NB,),
        in_specs=[
            pl.BlockSpec((512, 3), lambda i: (i, 0)),
            pl.BlockSpec((3, 32), lambda i: (0, 0)),
            pl.BlockSpec((3, 32), lambda i: (0, 0)),
        ],
        out_specs=[
            pl.BlockSpec((512, 32), lambda i: (i, 0)),
            pl.BlockSpec((512, 32), lambda i: (i, 0)),
        ],
        out_shape=[
            jax.ShapeDtypeStruct((NP, 32), jnp.float32),
            jax.ShapeDtypeStruct((NP, 32), jnp.float32),
        ],
    )(x_pad, wd1t, ws1t)


def _edge_prep(ea_pad, we1t, be1, we2t, be2, we3t, be3):
    blk = 8192
    nb = EP // blk

    def body(ea_ref, w1_ref, b1_ref, w2_ref, b2_ref, w3_ref, b3_ref,
             e1_ref, e2_ref, e3_ref):
        eb = ea_ref[...]
        bf = jnp.bfloat16
        e1_ref[...] = (_dot(eb, w1_ref[...]) + b1_ref[...]).astype(bf)
        e2_ref[0] = (_dot(eb, w2_ref[0]) + b2_ref[0]).astype(bf)
        e2_ref[1] = (_dot(eb, w2_ref[1]) + b2_ref[1]).astype(bf)
        e3_ref[0] = (_dot(eb, w3_ref[0]) + b3_ref[0]).astype(bf)
        e3_ref[1] = (_dot(eb, w3_ref[1]) + b3_ref[1]).astype(bf)

    e1, e2, e3 = pl.pallas_call(
        body,
        grid=(nb,),
        in_specs=[
            pl.BlockSpec((blk, 16), lambda i: (i, 0)),
            pl.BlockSpec((16, 32), lambda i: (0, 0)),
            pl.BlockSpec((1, 32), lambda i: (0, 0)),
            pl.BlockSpec((2, 16, 64), lambda i: (0, 0, 0)),
            pl.BlockSpec((2, 1, 64), lambda i: (0, 0, 0)),
            pl.BlockSpec((2, 16, 64), lambda i: (0, 0, 0)),
            pl.BlockSpec((2, 1, 64), lambda i: (0, 0, 0)),
        ],
        out_specs=[
            pl.BlockSpec((blk, 32), lambda i: (i, 0)),
            pl.BlockSpec((2, blk, 64), lambda i: (0, i, 0)),
            pl.BlockSpec((2, blk, 64), lambda i: (0, i, 0)),
        ],
        out_shape=[
            jax.ShapeDtypeStruct((EP, 32), jnp.bfloat16),
            jax.ShapeDtypeStruct((2, EP, 64), jnp.bfloat16),
            jax.ShapeDtypeStruct((2, EP, 64), jnp.bfloat16),
        ],
    )(ea_pad, we1t, be1, we2t, be2, we3t, be3)
    return e1, e2.reshape(2 * EP, 64), e3.reshape(2 * EP, 64)


def _mid1(x_pad, out1, wpt, bp, wd2t, ws2t):
    def body(x_ref, a0_ref, a1_ref, wp_ref, bp_ref, wd_ref, ws_ref,
             h_ref, d_ref, s_ref):
        aggr = a0_ref[...][:, :3] + a1_ref[...][:, :3]
        h1 = x_ref[...] + aggr
        h = jnp.maximum(_dot(h1, wp_ref[...]) + bp_ref[...], 0.0)
        h_ref[...] = h
        d_ref[...] = _dot(h, wd_ref[0])
        s_ref[...] = _dot(h, ws_ref[0])

    return pl.pallas_call(
        body,
        grid=(2, NB),
        in_specs=[
            pl.BlockSpec((512, 3), lambda c, i: (i, 0)),
            pl.BlockSpec((512, 16), lambda c, i: (i, 0)),
            pl.BlockSpec((512, 16), lambda c, i: (NB + i, 0)),
            pl.BlockSpec((3, 64), lambda c, i: (0, 0)),
            pl.BlockSpec((1, 64), lambda c, i: (0, 0)),
            pl.BlockSpec((1, 64, 64), lambda c, i: (c, 0, 0)),
            pl.BlockSpec((1, 64, 64), lambda c, i: (c, 0, 0)),
        ],
        out_specs=[
            pl.BlockSpec((512, 64), lambda c, i: (i, 0)),
            pl.BlockSpec((512, 64), lambda c, i: (c * NB + i, 0)),
            pl.BlockSpec((512, 64), lambda c, i: (c * NB + i, 0)),
        ],
        out_shape=[
            jax.ShapeDtypeStruct((NP, 64), jnp.float32),
            jax.ShapeDtypeStruct((2 * NP, 64), jnp.float32),
            jax.ShapeDtypeStruct((2 * NP, 64), jnp.float32),
        ],
    )(x_pad, out1, out1, wpt, bp, wd2t, ws2t)


def _mid2(h, out2, wd3t, ws3t):
    def body(h_ref, a0_ref, a1_ref, wd_ref, ws_ref, h2_ref, d_ref, s_ref):
        aggr = jnp.concatenate([a0_ref[...], a1_ref[...]], axis=1)
        h2 = jnp.maximum(h_ref[...] + aggr, 0.0)
        h2_ref[...] = h2
        d_ref[...] = _dot(h2, wd_ref[0])
        s_ref[...] = _dot(h2, ws_ref[0])

    return pl.pallas_call(
        body,
        grid=(2, NB),
        in_specs=[
            pl.BlockSpec((512, 64), lambda c, i: (i, 0)),
            pl.BlockSpec((512, 32), lambda c, i: (i, 0)),
            pl.BlockSpec((512, 32), lambda c, i: (NB + i, 0)),
            pl.BlockSpec((1, 64, 64), lambda c, i: (c, 0, 0)),
            pl.BlockSpec((1, 64, 64), lambda c, i: (c, 0, 0)),
        ],
        out_specs=[
            pl.BlockSpec((512, 64), lambda c, i: (i, 0)),
            pl.BlockSpec((512, 64), lambda c, i: (c * NB + i, 0)),
            pl.BlockSpec((512, 64), lambda c, i: (c * NB + i, 0)),
        ],
        out_shape=[
            jax.ShapeDtypeStruct((NP, 64), jnp.float32),
            jax.ShapeDtypeStruct((2 * NP, 64), jnp.float32),
            jax.ShapeDtypeStruct((2 * NP, 64), jnp.float32),
        ],
    )(h, out2, out2, wd3t, ws3t)


def _final(h2, out3, batch3, w1t, b1, w2t, b2):
    def body(h_ref, a0_ref, a1_ref, bt_ref, w1_ref, b1_ref, w2_ref, b2_ref,
             o_ref, sums, cnt):
        i = pl.program_id(0)

        @pl.when(i == 0)
        def _init():
            sums[...] = jnp.zeros_like(sums)
            cnt[...] = jnp.zeros_like(cnt)

        rowid = i * 512 + lax.broadcasted_iota(jnp.int32, (512, 1), 0)
        mask = (rowid < N).astype(jnp.float32)
        aggr = jnp.concatenate([a0_ref[...], a1_ref[...]], axis=1)
        h3 = jnp.maximum(h_ref[...] + aggr, 0.0) * mask
        b = bt_ref[0, 0, :]
        seg = lax.broadcasted_iota(jnp.int32, (512, G), 1)
        onehot = (b[:, None] == seg).astype(jnp.float32) * mask
        dotT = lambda a, b: jax.lax.dot_general(
            a, b, (((0,), (0,)), ((), ())), preferred_element_type=jnp.float32)
        sums[...] += dotT(onehot, h3)
        cnt[...] += dotT(onehot, jnp.broadcast_to(mask, (512, 8)))

        @pl.when(i == NB - 1)
        def _fin():
            pooled = sums[...] / jnp.maximum(cnt[...][:, 0:1], 1.0)
            r1 = jnp.maximum(_dot(pooled, w1_ref[...]) + b1_ref[...], 0.0)
            o_ref[...] = _dot(r1, w2_ref[...]) + b2_ref[...]

    return pl.pallas_call(
        body,
        grid=(NB,),
        in_specs=[
            pl.BlockSpec((512, 64), lambda i: (i, 0)),
            pl.BlockSpec((512, 32), lambda i: (i, 0)),
            pl.BlockSpec((512, 32), lambda i: (NB + i, 0)),
            pl.BlockSpec((1, 1, 512), lambda i: (i, 0, 0)),
            pl.BlockSpec((64, 64), lambda i: (0, 0)),
            pl.BlockSpec((1, 64), lambda i: (0, 0)),
            pl.BlockSpec((64, 3), lambda i: (0, 0)),
            pl.BlockSpec((1, 3), lambda i: (0, 0)),
        ],
        out_specs=pl.BlockSpec((G, 3), lambda i: (0, 0)),
        out_shape=jax.ShapeDtypeStruct((G, 3), jnp.float32),
        scratch_shapes=[
            pltpu.VMEM((G, 64), jnp.float32),
            pltpu.VMEM((G, 8), jnp.float32),
        ],
    )(h2, out3, out3, batch3, w1t, b1, w2t, b2)


def kernel(x, edge_index, edge_attr, batch,
           Wf1, bf1, Ws1, bs1, Wp, bp,
           Wf2, bf2, Ws2, bs2, Wf3, bf3, Ws3, bs3,
           W1, b1, W2, b2):
    f32 = jnp.float32
    src = edge_index[0].astype(jnp.int32)
    dst = edge_index[1].astype(jnp.int32)

    x_pad = jnp.pad(x, ((0, NP - N), (0, 0)))
    padv = N + (jnp.arange(EP - E, dtype=jnp.int32) % 16)
    dst_pad = jnp.concatenate([dst, padv])
    src_pad = jnp.concatenate([src, jnp.zeros((EP - E,), jnp.int32)])
    dg2 = jnp.concatenate([dst_pad, dst_pad + NP])
    sg2 = jnp.concatenate([src_pad, src_pad + NP])
    batch3 = jnp.pad(batch.astype(jnp.int32), (0, NP - N)).reshape(NB, 1, 512)
    zz16 = jnp.zeros((NPT, 16), f32)
    zz32 = jnp.zeros((NPT, 32), f32)

    wd1 = jnp.zeros((32, 3), f32).at[0:3].set(Wf1[:, 0:3]).at[16:19].set(Ws1[:, 0:3])
    ws1 = jnp.zeros((32, 3), f32).at[0:3].set(Wf1[:, 3:6]).at[16:19].set(Ws1[:, 3:6])
    we1 = jnp.zeros((32, 16), f32).at[0:3].set(Wf1[:, 6:22]).at[16:19].set(Ws1[:, 6:22])
    be1 = jnp.zeros((32,), f32).at[0:3].set(bf1).at[16:19].set(bs1)

    def split23(Wf, bf, Ws, bs):
        wd = jnp.stack([jnp.concatenate([Wf[0:32, 0:64], Ws[0:32, 0:64]]),
                        jnp.concatenate([Wf[32:64, 0:64], Ws[32:64, 0:64]])])
        ws_ = jnp.stack([jnp.concatenate([Wf[0:32, 64:128], Ws[0:32, 64:128]]),
                         jnp.concatenate([Wf[32:64, 64:128], Ws[32:64, 64:128]])])
        we = jnp.stack([jnp.concatenate([Wf[0:32, 128:144], Ws[0:32, 128:144]]),
                        jnp.concatenate([Wf[32:64, 128:144], Ws[32:64, 128:144]])])
        be = jnp.stack([jnp.concatenate([bf[0:32], bs[0:32]]),
                        jnp.concatenate([bf[32:64], bs[32:64]])])
        return wd, ws_, we, be

    wd2, ws2, we2, be2 = split23(Wf2, bf2, Ws2, bs2)
    wd3, ws3, we3, be3 = split23(Wf3, bf3, Ws3, bs3)

    i16 = jnp.arange(16, dtype=jnp.int32)
    p32 = jnp.stack([i16, i16 + 16], axis=1).reshape(32)
    p64 = jnp.concatenate([p32, p32 + 32])
    we1 = we1[p32]
    be1 = be1[p32]
    we2 = we2[:, p64]
    be2 = be2[:, p64]
    we3 = we3[:, p64]
    be3 = be3[:, p64]

    d1, s1 = _node_prep1(x_pad, wd1.T, ws1.T)
    e1, e2, e3 = _edge_prep(edge_attr, we1.T, be1[None, :],
                            jnp.transpose(we2, (0, 2, 1)), be2[:, None, :],
                            jnp.transpose(we3, (0, 2, 1)), be3[:, None, :])

    e1 = lax.bitcast_convert_type(e1.reshape(EP, 16, 2), jnp.int32)
    e2 = lax.bitcast_convert_type(e2.reshape(2 * EP, 32, 2), jnp.int32)
    e3 = lax.bitcast_convert_type(e3.reshape(2 * EP, 32, 2), jnp.int32)

    out1 = _sc_edge_layer(32, 392, EP // 2, 0, 0, d1, s1, e1,
                          dst_pad, src_pad, dst_pad, zz16)

    h, d2, s2 = _mid1(x_pad, out1, Wp.T, bp[None, :],
                      jnp.transpose(wd2, (0, 2, 1)), jnp.transpose(ws2, (0, 2, 1)))
    d2 = d2.reshape(2 * NP, 64)
    s2 = s2.reshape(2 * NP, 64)

    out2 = _sc_edge_layer(64, 784, 0, EP, EP, d2, s2, e2,
                          dg2, sg2, dst_pad, zz32)

    h2, d3, s3 = _mid2(h, out2, jnp.transpose(wd3, (0, 2, 1)),
                       jnp.transpose(ws3, (0, 2, 1)))

    out3 = _sc_edge_layer(64, 784, 0, EP, EP, d3, s3, e3,
                          dg2, sg2, dst_pad, zz32)

    return _final(h2, out3, batch3, W1.T, b1[None, :], W2.T, b2[None, :])


# speedup vs baseline: 2.1944x; 2.1944x over previous
import functools

import jax
import jax.numpy as jnp
from jax import lax
from jax.experimental import pallas as pl
from jax.experimental.pallas import tpu as pltpu
from jax.experimental.pallas import tpu_sc as plsc

N = 50000
E = 800000
G = 64
NP = 50176
EP = 802816
NB = NP // 512
EB = EP // 512
NPT = NP // 16

C1 = 0.9864523801332455
C2 = -0.40368039920027776
C3 = 0.1105132845506506


def _log1p16(t):
    return t * (C1 + t * (C2 + t * C3))


def _gate16(a, b):
    ta = jnp.exp(-jnp.abs(a))
    tb = jnp.exp(-jnp.abs(b))
    sp_b = jnp.maximum(b, 0.0) + _log1p16(tb)
    sig_a = jnp.exp(jnp.minimum(a, 0.0) - _log1p16(ta))
    return sig_a * sp_b


def _sc_edge_layer(tw, cpt, ebase_c, idxoff_c, eoff_c, d_tbl, s_tbl, e_arr,
                   dg, sg, ds, zz):
    mw = tw // 2
    ck = 64
    mesh = plsc.VectorSubcoreMesh(core_axis_name="c", subcore_axis_name="s")

    @functools.partial(
        pl.kernel,
        mesh=mesh,
        compiler_params=pltpu.CompilerParams(use_tc_tiling_on_sc=False),
        out_type=jax.ShapeDtypeStruct((2 * NP, mw), jnp.float32),
        scratch_types=[
            pltpu.VMEM((3, ck), jnp.int32),
            pltpu.VMEM((3, ck), jnp.int32),
            pltpu.VMEM((3, ck), jnp.int32),
            pltpu.VMEM((2, ck, tw), jnp.float32),
            pltpu.VMEM((2, ck, tw), jnp.float32),
            pltpu.VMEM((2, ck, tw // 2), jnp.int32),
            pltpu.VMEM((2, ck, mw), jnp.float32),
            pltpu.VMEM_SHARED((NP, mw), jnp.float32),
            pltpu.SemaphoreType.DMA,
            pltpu.SemaphoreType.DMA,
            pltpu.SemaphoreType.DMA,
            pltpu.SemaphoreType.DMA,
            pltpu.SemaphoreType.DMA,
        ],
    )
    def k(d_hbm, s_hbm, e_hbm, dg_hbm, sg_hbm, ds_hbm, zz_hbm, out_hbm,
          dgi_v, sgi_v, dsi_v, rows_d, rows_s, rows_e, m_v, acc,
          sem_d, sem_s, sem_e, sem_i, sem_m):
        c = lax.axis_index("c")
        s = lax.axis_index("s")
        pltpu.sync_copy(zz_hbm, acc.at[pl.ds(s * NPT, NPT)])
        plsc.subcore_barrier()

        base = ebase_c * c + s * (cpt * ck)

        def idx_issue(i):
            e0 = base + i * ck
            eg = idxoff_c * c + e0
            p = lax.rem(i, 3)
            pltpu.async_copy(dg_hbm.at[pl.ds(eg, ck)], dgi_v.at[p], sem_i)
            pltpu.async_copy(sg_hbm.at[pl.ds(eg, ck)], sgi_v.at[p], sem_i)
            pltpu.async_copy(ds_hbm.at[pl.ds(e0, ck)], dsi_v.at[p], sem_i)

        def idx_wait(i):
            p = lax.rem(i, 3)
            pltpu.make_async_copy(dg_hbm.at[pl.ds(0, ck)], dgi_v.at[p], sem_i).wait()
            pltpu.make_async_copy(sg_hbm.at[pl.ds(0, ck)], sgi_v.at[p], sem_i).wait()
            pltpu.make_async_copy(ds_hbm.at[pl.ds(0, ck)], dsi_v.at[p], sem_i).wait()

        def rows_issue(i):
            e0 = base + i * ck
            p = lax.rem(i, 2)
            pi = lax.rem(i, 3)
            pltpu.async_copy(d_hbm.at[dgi_v.at[pi]], rows_d.at[p], sem_d)
            pltpu.async_copy(s_hbm.at[sgi_v.at[pi]], rows_s.at[p], sem_s)
            pltpu.async_copy(e_hbm.at[pl.ds(eoff_c * c + e0, ck)],
                             rows_e.at[p], sem_e)

        def rows_wait(i):
            p = lax.rem(i, 2)
            pltpu.make_async_copy(e_hbm.at[pl.ds(0, ck)], rows_d.at[p], sem_d).wait()
            pltpu.make_async_copy(e_hbm.at[pl.ds(0, ck)], rows_s.at[p], sem_s).wait()
            pltpu.make_async_copy(e_hbm.at[pl.ds(0, ck)], rows_e.at[p], sem_e).wait()

        idx_issue(0)
        idx_wait(0)
        rows_issue(0)
        idx_issue(1)

        def chunk(i, carry):
            rows_wait(i)

            @pl.when(i > 0)
            def _w_sc():
                pltpu.make_async_copy(
                    zz_hbm.at[pl.ds(0, ck)], m_v.at[0], sem_m).wait()

            @pl.when(i + 1 < cpt)
            def _pf_rows():
                idx_wait(i + 1)
                rows_issue(i + 1)

            @pl.when(i + 2 < cpt)
            def _pf_idx():
                idx_issue(i + 2)

            p = lax.rem(i, 2)
            pi = lax.rem(i, 3)

            @plsc.parallel_loop(0, ck, unroll=8)
            def row(r):
                for j in range(tw // 32):
                    oa = 16 * j
                    ob = mw + 16 * j
                    w = rows_e[p, r, pl.ds(16 * j, 16)]
                    elo = lax.bitcast_convert_type(
                        lax.shift_left(w, 16), jnp.float32)
                    ehi = lax.bitcast_convert_type(
                        lax.bitwise_and(w, jnp.int32(-65536)), jnp.float32)
                    a = (rows_d[p, r, pl.ds(oa, 16)] + rows_s[p, r, pl.ds(oa, 16)]
                         + elo)
                    b = (rows_d[p, r, pl.ds(ob, 16)] + rows_s[p, r, pl.ds(ob, 16)]
                         + ehi)
                    m_v[p, r, pl.ds(oa, 16)] = _gate16(a, b)
            pltpu.async_copy(m_v.at[p], acc.at[dsi_v.at[pi]], sem_m, add=True)
            return carry

        lax.fori_loop(0, cpt, chunk, 0)
        pltpu.make_async_copy(zz_hbm.at[pl.ds(0, ck)], m_v.at[0], sem_m).wait()
        plsc.subcore_barrier()
        pltpu.sync_copy(acc.at[pl.ds(s * NPT, NPT)],
                        out_hbm.at[pl.ds(c * NP + s * NPT, NPT)])

    return k(d_tbl, s_tbl, e_arr, dg, sg, ds, zz)


def _dot(a, b):
    return jax.lax.dot_general(a, b, (((1,), (0,)), ((), ())),
                               preferred_element_type=jnp.float32)


def _node_prep1(x_pad, wd1t, ws1t):
    def body(x_ref, wd_ref, ws_ref, d_ref, s_ref):
        xb = x_ref[...]
        d_ref[...] = _dot(xb, wd_ref[...])
        s_ref[...] = _dot(xb, ws_ref[...])

    return pl.pallas_call(
        body,
        grid=(NB,),
        in_specs=[
            pl.BlockSpec((512, 3), lambda i: (i, 0)),
            pl.BlockSpec((3, 32), lambda i: (0, 0)),
            pl.BlockSpec((3, 32), lambda i: (0, 0)),
        ],
        out_specs=[
            pl.BlockSpec((512, 32), lambda i: (i, 0)),
            pl.BlockSpec((512, 32), lambda i: (i, 0)),
        ],
        out_shape=[
            jax.ShapeDtypeStruct((NP, 32), jnp.float32),
            jax.ShapeDtypeStruct((NP, 32), jnp.float32),
        ],
    )(x_pad, wd1t, ws1t)


def _edge_prep(ea_pad, we1t, be1, we2t, be2, we3t, be3):
    blk = 8192
    nb = EP // blk

    def body(ea_ref, w1_ref, b1_ref, w2_ref, b2_ref, w3_ref, b3_ref,
             e1_ref, e2_ref, e3_ref):
        eb = ea_ref[...]

        def pack(z):
            u = lax.bitcast_convert_type(z, jnp.int32)
            v = lax.shift_right_logical(
                u + 0x7FFF + lax.bitwise_and(
                    lax.shift_right_logical(u, 16), 1), 16)
            h = z.shape[1] // 2
            return lax.bitwise_or(v[:, :h],
                                  lax.shift_left(v[:, h:], 16))

        e1_ref[...] = pack(_dot(eb, w1_ref[...]) + b1_ref[...])
        e2_ref[0] = pack(_dot(eb, w2_ref[0]) + b2_ref[0])
        e2_ref[1] = pack(_dot(eb, w2_ref[1]) + b2_ref[1])
        e3_ref[0] = pack(_dot(eb, w3_ref[0]) + b3_ref[0])
        e3_ref[1] = pack(_dot(eb, w3_ref[1]) + b3_ref[1])

    e1, e2, e3 = pl.pallas_call(
        body,
        grid=(nb,),
        in_specs=[
            pl.BlockSpec((blk, 16), lambda i: (i, 0)),
            pl.BlockSpec((16, 32), lambda i: (0, 0)),
            pl.BlockSpec((1, 32), lambda i: (0, 0)),
            pl.BlockSpec((2, 16, 64), lambda i: (0, 0, 0)),
            pl.BlockSpec((2, 1, 64), lambda i: (0, 0, 0)),
            pl.BlockSpec((2, 16, 64), lambda i: (0, 0, 0)),
            pl.BlockSpec((2, 1, 64), lambda i: (0, 0, 0)),
        ],
        out_specs=[
            pl.BlockSpec((blk, 16), lambda i: (i, 0)),
            pl.BlockSpec((2, blk, 32), lambda i: (0, i, 0)),
            pl.BlockSpec((2, blk, 32), lambda i: (0, i, 0)),
        ],
        out_shape=[
            jax.ShapeDtypeStruct((EP, 16), jnp.int32),
            jax.ShapeDtypeStruct((2, EP, 32), jnp.int32),
            jax.ShapeDtypeStruct((2, EP, 32), jnp.int32),
        ],
    )(ea_pad, we1t, be1, we2t, be2, we3t, be3)
    return e1, e2.reshape(2 * EP, 32), e3.reshape(2 * EP, 32)


def _mid1(x_pad, out1, wpt, bp, wd2t, ws2t):
    def body(x_ref, a0_ref, a1_ref, wp_ref, bp_ref, wd_ref, ws_ref,
             h_ref, d_ref, s_ref):
        aggr = a0_ref[...][:, :3] + a1_ref[...][:, :3]
        h1 = x_ref[...] + aggr
        h = jnp.maximum(_dot(h1, wp_ref[...]) + bp_ref[...], 0.0)
        h_ref[...] = h
        d_ref[...] = _dot(h, wd_ref[0])
        s_ref[...] = _dot(h, ws_ref[0])

    return pl.pallas_call(
        body,
        grid=(2, NB),
        in_specs=[
            pl.BlockSpec((512, 3), lambda c, i: (i, 0)),
            pl.BlockSpec((512, 16), lambda c, i: (i, 0)),
            pl.BlockSpec((512, 16), lambda c, i: (NB + i, 0)),
            pl.BlockSpec((3, 64), lambda c, i: (0, 0)),
            pl.BlockSpec((1, 64), lambda c, i: (0, 0)),
            pl.BlockSpec((1, 64, 64), lambda c, i: (c, 0, 0)),
            pl.BlockSpec((1, 64, 64), lambda c, i: (c, 0, 0)),
        ],
        out_specs=[
            pl.BlockSpec((512, 64), lambda c, i: (i, 0)),
            pl.BlockSpec((512, 64), lambda c, i: (c * NB + i, 0)),
            pl.BlockSpec((512, 64), lambda c, i: (c * NB + i, 0)),
        ],
        out_shape=[
            jax.ShapeDtypeStruct((NP, 64), jnp.float32),
            jax.ShapeDtypeStruct((2 * NP, 64), jnp.float32),
            jax.ShapeDtypeStruct((2 * NP, 64), jnp.float32),
        ],
    )(x_pad, out1, out1, wpt, bp, wd2t, ws2t)


def _mid2(h, out2, wd3t, ws3t):
    def body(h_ref, a0_ref, a1_ref, wd_ref, ws_ref, h2_ref, d_ref, s_ref):
        aggr = jnp.concatenate([a0_ref[...], a1_ref[...]], axis=1)
        h2 = jnp.maximum(h_ref[...] + aggr, 0.0)
        h2_ref[...] = h2
        d_ref[...] = _dot(h2, wd_ref[0])
        s_ref[...] = _dot(h2, ws_ref[0])

    return pl.pallas_call(
        body,
        grid=(2, NB),
        in_specs=[
            pl.BlockSpec((512, 64), lambda c, i: (i, 0)),
            pl.BlockSpec((512, 32), lambda c, i: (i, 0)),
            pl.BlockSpec((512, 32), lambda c, i: (NB + i, 0)),
            pl.BlockSpec((1, 64, 64), lambda c, i: (c, 0, 0)),
            pl.BlockSpec((1, 64, 64), lambda c, i: (c, 0, 0)),
        ],
        out_specs=[
            pl.BlockSpec((512, 64), lambda c, i: (i, 0)),
            pl.BlockSpec((512, 64), lambda c, i: (c * NB + i, 0)),
            pl.BlockSpec((512, 64), lambda c, i: (c * NB + i, 0)),
        ],
        out_shape=[
            jax.ShapeDtypeStruct((NP, 64), jnp.float32),
            jax.ShapeDtypeStruct((2 * NP, 64), jnp.float32),
            jax.ShapeDtypeStruct((2 * NP, 64), jnp.float32),
        ],
    )(h, out2, out2, wd3t, ws3t)


def _final(h2, out3, batch3, w1t, b1, w2t, b2):
    def body(h_ref, a0_ref, a1_ref, bt_ref, w1_ref, b1_ref, w2_ref, b2_ref,
             o_ref, sums, cnt):
        i = pl.program_id(0)

        @pl.when(i == 0)
        def _init():
            sums[...] = jnp.zeros_like(sums)
            cnt[...] = jnp.zeros_like(cnt)

        rowid = i * 512 + lax.broadcasted_iota(jnp.int32, (512, 1), 0)
        mask = (rowid < N).astype(jnp.float32)
        aggr = jnp.concatenate([a0_ref[...], a1_ref[...]], axis=1)
        h3 = jnp.maximum(h_ref[...] + aggr, 0.0) * mask
        b = bt_ref[0, 0, :]
        seg = lax.broadcasted_iota(jnp.int32, (512, G), 1)
        onehot = (b[:, None] == seg).astype(jnp.float32) * mask
        dotT = lambda a, b: jax.lax.dot_general(
            a, b, (((0,), (0,)), ((), ())), preferred_element_type=jnp.float32)
        sums[...] += dotT(onehot, h3)
        cnt[...] += dotT(onehot, jnp.broadcast_to(mask, (512, 8)))

        @pl.when(i == NB - 1)
        def _fin():
            pooled = sums[...] / jnp.maximum(cnt[...][:, 0:1], 1.0)
            r1 = jnp.maximum(_dot(pooled, w1_ref[...]) + b1_ref[...], 0.0)
            o_ref[...] = _dot(r1, w2_ref[...]) + b2_ref[...]

    return pl.pallas_call(
        body,
        grid=(NB,),
        in_specs=[
            pl.BlockSpec((512, 64), lambda i: (i, 0)),
            pl.BlockSpec((512, 32), lambda i: (i, 0)),
            pl.BlockSpec((512, 32), lambda i: (NB + i, 0)),
            pl.BlockSpec((1, 1, 512), lambda i: (i, 0, 0)),
            pl.BlockSpec((64, 64), lambda i: (0, 0)),
            pl.BlockSpec((1, 64), lambda i: (0, 0)),
            pl.BlockSpec((64, 3), lambda i: (0, 0)),
            pl.BlockSpec((1, 3), lambda i: (0, 0)),
        ],
        out_specs=pl.BlockSpec((G, 3), lambda i: (0, 0)),
        out_shape=jax.ShapeDtypeStruct((G, 3), jnp.float32),
        scratch_shapes=[
            pltpu.VMEM((G, 64), jnp.float32),
            pltpu.VMEM((G, 8), jnp.float32),
        ],
    )(h2, out3, out3, batch3, w1t, b1, w2t, b2)


def kernel(x, edge_index, edge_attr, batch,
           Wf1, bf1, Ws1, bs1, Wp, bp,
           Wf2, bf2, Ws2, bs2, Wf3, bf3, Ws3, bs3,
           W1, b1, W2, b2):
    f32 = jnp.float32
    src = edge_index[0].astype(jnp.int32)
    dst = edge_index[1].astype(jnp.int32)

    x_pad = jnp.pad(x, ((0, NP - N), (0, 0)))
    padv = N + (jnp.arange(EP - E, dtype=jnp.int32) % 16)
    dst_pad = jnp.concatenate([dst, padv])
    src_pad = jnp.concatenate([src, jnp.zeros((EP - E,), jnp.int32)])
    dg2 = jnp.concatenate([dst_pad, dst_pad + NP])
    sg2 = jnp.concatenate([src_pad, src_pad + NP])
    batch3 = jnp.pad(batch.astype(jnp.int32), (0, NP - N)).reshape(NB, 1, 512)
    zz16 = jnp.zeros((NPT, 16), f32)
    zz32 = jnp.zeros((NPT, 32), f32)

    wd1 = jnp.zeros((32, 3), f32).at[0:3].set(Wf1[:, 0:3]).at[16:19].set(Ws1[:, 0:3])
    ws1 = jnp.zeros((32, 3), f32).at[0:3].set(Wf1[:, 3:6]).at[16:19].set(Ws1[:, 3:6])
    we1 = jnp.zeros((32, 16), f32).at[0:3].set(Wf1[:, 6:22]).at[16:19].set(Ws1[:, 6:22])
    be1 = jnp.zeros((32,), f32).at[0:3].set(bf1).at[16:19].set(bs1)

    def split23(Wf, bf, Ws, bs):
        wd = jnp.stack([jnp.concatenate([Wf[0:32, 0:64], Ws[0:32, 0:64]]),
                        jnp.concatenate([Wf[32:64, 0:64], Ws[32:64, 0:64]])])
        ws_ = jnp.stack([jnp.concatenate([Wf[0:32, 64:128], Ws[0:32, 64:128]]),
                         jnp.concatenate([Wf[32:64, 64:128], Ws[32:64, 64:128]])])
        we = jnp.stack([jnp.concatenate([Wf[0:32, 128:144], Ws[0:32, 128:144]]),
                        jnp.concatenate([Wf[32:64, 128:144], Ws[32:64, 128:144]])])
        be = jnp.stack([jnp.concatenate([bf[0:32], bs[0:32]]),
                        jnp.concatenate([bf[32:64], bs[32:64]])])
        return wd, ws_, we, be

    wd2, ws2, we2, be2 = split23(Wf2, bf2, Ws2, bs2)
    wd3, ws3, we3, be3 = split23(Wf3, bf3, Ws3, bs3)

    d1, s1 = _node_prep1(x_pad, wd1.T, ws1.T)
    e1, e2, e3 = _edge_prep(edge_attr, we1.T, be1[None, :],
                            jnp.transpose(we2, (0, 2, 1)), be2[:, None, :],
                            jnp.transpose(we3, (0, 2, 1)), be3[:, None, :])

    out1 = _sc_edge_layer(32, 392, EP // 2, 0, 0, d1, s1, e1,
                          dst_pad, src_pad, dst_pad, zz16)

    h, d2, s2 = _mid1(x_pad, out1, Wp.T, bp[None, :],
                      jnp.transpose(wd2, (0, 2, 1)), jnp.transpose(ws2, (0, 2, 1)))
    d2 = d2.reshape(2 * NP, 64)
    s2 = s2.reshape(2 * NP, 64)

    out2 = _sc_edge_layer(64, 784, 0, EP, EP, d2, s2, e2,
                          dg2, sg2, dst_pad, zz32)

    h2, d3, s3 = _mid2(h, out2, jnp.transpose(wd3, (0, 2, 1)),
                       jnp.transpose(ws3, (0, 2, 1)))

    out3 = _sc_edge_layer(64, 784, 0, EP, EP, d3, s3, e3,
                          dg2, sg2, dst_pad, zz32)

    return _final(h2, out3, batch3, W1.T, b1[None, :], W2.T, b2[None, :])
